# baseline (speedup 1.0000x reference)
import functools

import jax
import jax.numpy as jnp
from jax import lax
from jax.experimental import pallas as pl
from jax.experimental.pallas import tpu as pltpu
from jax.experimental.pallas import tpu_sc as plsc

B = 2
C = 8
D = 64
H = 192
W = 192
SC_D = 24
TC_D = D - SC_D
DC = 8
NW = 32
RCH = 24
NRC = H // RCH
UNITS = B * SC_D * NRC
UPW = UNITS // NW
LANES = 16
NKEY = B * C * C
ACC_LEN = NKEY * LANES

_mesh = plsc.VectorSubcoreMesh(core_axis_name="c", subcore_axis_name="s")


@functools.partial(
    pl.kernel,
    mesh=_mesh,
    out_type=jax.ShapeDtypeStruct((NW, ACC_LEN), jnp.float32),
    compiler_params=pltpu.CompilerParams(needs_layout_passes=False),
    scratch_types=[
        pltpu.VMEM((2, C, RCH, W), jnp.float32),
        pltpu.VMEM((2, RCH, W), jnp.int32),
        pltpu.VMEM((ACC_LEN,), jnp.float32),
        pltpu.SemaphoreType.DMA,
        pltpu.SemaphoreType.DMA,
    ],
)
def _sc_hist(x_hbm, t_hbm, out_hbm, inb, tgb, acc, sem0, sem1):
    nc = 2
    wid = lax.axis_index("s") * nc + lax.axis_index("c")
    ubase = wid * UPW
    sems = (sem0, sem1)

    zero16 = jnp.zeros((LANES,), jnp.float32)

    def _zero(i, carry):
        acc[pl.ds(i * LANES, LANES)] = zero16
        return carry

    lax.fori_loop(0, NKEY, _zero, 0)

    def _unit_coords(u):
        unit = ubase + u
        b = unit // (SC_D * NRC)
        rem = unit % (SC_D * NRC)
        d = rem // NRC
        r0 = (rem % NRC) * RCH
        return b, d, r0

    def _start(u, slot):
        b, d, r0 = _unit_coords(u)
        pltpu.async_copy(
            x_hbm.at[b, :, d, pl.ds(r0, RCH), :], inb.at[slot], sems[slot])
        pltpu.async_copy(
            t_hbm.at[b, d, pl.ds(r0, RCH), :], tgb.at[slot], sems[slot])

    def _wait(slot):
        pltpu.make_async_copy(
            x_hbm.at[0, :, 0, pl.ds(0, RCH), :], inb.at[slot],
            sems[slot]).wait()
        pltpu.make_async_copy(
            t_hbm.at[0, 0, pl.ds(0, RCH), :], tgb.at[slot], sems[slot]).wait()

    lane = lax.iota(jnp.int32, LANES)
    ones = jnp.ones((LANES,), jnp.float32)
    npr = W // LANES

    def _compute(u, slot):
        b, _, _ = _unit_coords(u)
        kbase = b * (C * C * LANES)

        def body(r, carry):
            for i in range(npr):
                p = i * LANES
                best = inb[slot, 0, r, pl.ds(p, LANES)]
                idx = jnp.zeros((LANES,), jnp.int32)
                for c in range(1, C):
                    v = inb[slot, c, r, pl.ds(p, LANES)]
                    m = v > best
                    best = jnp.where(m, v, best)
                    idx = jnp.where(m, jnp.full((LANES,), c, jnp.int32), idx)
                t = tgb[slot, r, pl.ds(p, LANES)]
                addr = kbase + (idx * C + t) * LANES + lane
                plsc.addupdate_scatter(acc, [addr], ones)
            return carry

        lax.fori_loop(0, RCH, body, 0)

    _start(0, 0)

    def outer(k, carry):
        _start(2 * k + 1, 1)
        _wait(0)
        _compute(2 * k, 0)

        @pl.when(k < UPW // 2 - 1)
        def _():
            _start(2 * k + 2, 0)

        _wait(1)
        _compute(2 * k + 1, 1)
        return carry

    lax.fori_loop(0, UPW // 2, outer, 0)

    pltpu.sync_copy(acc, out_hbm.at[wid])


def _tc_hist_body(x_ref, t_ref, o_ref):
    bi = pl.program_id(0)
    j = pl.program_id(1)

    @pl.when((bi == 0) & (j == 0))
    def _():
        o_ref[...] = jnp.zeros_like(o_ref)

    x = x_ref[0]
    t = t_ref[0]

    best = x[0]
    idx = jnp.zeros(best.shape, jnp.int32)
    for c in range(1, C):
        v = x[c]
        m = v > best
        best = jnp.where(m, v, best)
        idx = jnp.where(m, c, idx)

    rows = lax.broadcasted_iota(jnp.int32, (16, 128), 0)
    lanes = lax.broadcasted_iota(jnp.int32, (16, 128), 1)
    acc = o_ref[...]
    for c in range(1, C):
        pc = idx == c
        tc = t == c
        tps = jnp.sum(jnp.where(pc & tc, 1.0, 0.0))
        npred = jnp.sum(jnp.where(pc, 1.0, 0.0))
        nt = jnp.sum(jnp.where(tc, 1.0, 0.0))
        row = bi * 8 + c
        upd = jnp.where(lanes == 0, tps,
                        jnp.where(lanes == 1, npred,
                                  jnp.where(lanes == 2, nt, 0.0)))
        acc = jnp.where(rows == row, acc + upd, acc)
    o_ref[...] = acc


def _dice_body(p_ref, q_ref, o_ref):
    x = p_ref[:]
    tot = jnp.sum(x, axis=0, keepdims=True)
    l = lax.broadcasted_iota(jnp.int32, (1, ACC_LEN), 1)
    key = l // LANES
    b_l = key // (C * C)
    p_l = (key // C) % C
    t_l = key % C
    q = q_ref[:]
    eps = 1e-5
    res = []
    for b in range(B):
        mb = b_l == b
        s = 0.0
        for c in range(1, C):
            tps = (jnp.sum(jnp.where(mb & (p_l == c) & (t_l == c), tot, 0.0))
                   + q[b * 8 + c, 0])
            npred = (jnp.sum(jnp.where(mb & (p_l == c), tot, 0.0))
                     + q[b * 8 + c, 1])
            nt = (jnp.sum(jnp.where(mb & (t_l == c), tot, 0.0))
                  + q[b * 8 + c, 2])
            s = s + 2.0 * tps / (npred + nt + eps)
        res.append(s / (C - 1))
    rows = lax.broadcasted_iota(jnp.int32, (8, 128), 0)
    lanes = lax.broadcasted_iota(jnp.int32, (8, 128), 1)
    out = jnp.where((rows == 0) & (lanes == 0), res[0], 0.0)
    out = jnp.where((rows == 1) & (lanes == 0), res[1], out)
    o_ref[:] = out


def kernel(inputs, targets):
    t = targets.astype(jnp.int32)
    partial = _sc_hist(inputs, t)
    tc_counts = pl.pallas_call(
        _tc_hist_body,
        grid=(B, TC_D // DC),
        in_specs=[
            pl.BlockSpec((1, C, DC, H, W),
                         lambda b, j: (b, 0, SC_D // DC + j, 0, 0)),
            pl.BlockSpec((1, DC, H, W),
                         lambda b, j: (b, SC_D // DC + j, 0, 0)),
        ],
        out_specs=pl.BlockSpec((16, 128), lambda b, j: (0, 0)),
        out_shape=jax.ShapeDtypeStruct((16, 128), jnp.float32),
    )(inputs, t)
    out = pl.pallas_call(
        _dice_body,
        out_shape=jax.ShapeDtypeStruct((8, 128), jnp.float32),
    )(partial, tc_counts)
    return out[0:2, 0]


# speedup vs baseline: 1.0580x; 1.0580x over previous
import functools

import jax
import jax.numpy as jnp
from jax import lax
from jax.experimental import pallas as pl
from jax.experimental.pallas import tpu as pltpu
from jax.experimental.pallas import tpu_sc as plsc

B = 2
C = 8
D = 64
H = 192
W = 192
SC_D = 24
TC_D = D - SC_D
DC = 8
NW = 32
RCH = 24
NRC = H // RCH
UNITS = B * SC_D * NRC
UPW = UNITS // NW
LANES = 16
NKEY = B * C * C
ACC_LEN = NKEY * LANES

_mesh = plsc.VectorSubcoreMesh(core_axis_name="c", subcore_axis_name="s")


@functools.partial(
    pl.kernel,
    mesh=_mesh,
    out_type=jax.ShapeDtypeStruct((NW, ACC_LEN), jnp.float32),
    compiler_params=pltpu.CompilerParams(needs_layout_passes=False),
    scratch_types=[
        pltpu.VMEM((2, C, RCH, W), jnp.float32),
        pltpu.VMEM((2, RCH, W), jnp.int32),
        pltpu.VMEM((ACC_LEN,), jnp.float32),
        pltpu.SemaphoreType.DMA,
        pltpu.SemaphoreType.DMA,
    ],
)
def _sc_hist(x_hbm, t_hbm, out_hbm, inb, tgb, acc, sem0, sem1):
    nc = 2
    wid = lax.axis_index("s") * nc + lax.axis_index("c")
    ubase = wid * UPW
    sems = (sem0, sem1)

    zero16 = jnp.zeros((LANES,), jnp.float32)

    def _zero(i, carry):
        acc[pl.ds(i * LANES, LANES)] = zero16
        return carry

    lax.fori_loop(0, NKEY, _zero, 0)

    def _unit_coords(u):
        unit = ubase + u
        b = unit // (SC_D * NRC)
        rem = unit % (SC_D * NRC)
        d = rem // NRC
        r0 = (rem % NRC) * RCH
        return b, d, r0

    def _start(u, slot):
        b, d, r0 = _unit_coords(u)
        pltpu.async_copy(
            x_hbm.at[b, :, d, pl.ds(r0, RCH), :], inb.at[slot], sems[slot])
        pltpu.async_copy(
            t_hbm.at[b, d, pl.ds(r0, RCH), :], tgb.at[slot], sems[slot])

    def _wait(slot):
        pltpu.make_async_copy(
            x_hbm.at[0, :, 0, pl.ds(0, RCH), :], inb.at[slot],
            sems[slot]).wait()
        pltpu.make_async_copy(
            t_hbm.at[0, 0, pl.ds(0, RCH), :], tgb.at[slot], sems[slot]).wait()

    lane = lax.iota(jnp.int32, LANES)
    ones = jnp.ones((LANES,), jnp.float32)
    npr = W // LANES

    def _compute(u, slot):
        b, _, _ = _unit_coords(u)
        kbase = b * (C * C * LANES)

        def body(r, carry):
            for i in range(npr):
                p = i * LANES
                best = inb[slot, 0, r, pl.ds(p, LANES)]
                idx = jnp.zeros((LANES,), jnp.int32)
                for c in range(1, C):
                    v = inb[slot, c, r, pl.ds(p, LANES)]
                    m = v > best
                    best = jnp.where(m, v, best)
                    idx = jnp.where(m, jnp.full((LANES,), c, jnp.int32), idx)
                t = tgb[slot, r, pl.ds(p, LANES)]
                addr = kbase + (idx * C + t) * LANES + lane
                plsc.addupdate_scatter(acc, [addr], ones)
            return carry

        lax.fori_loop(0, RCH, body, 0)

    _start(0, 0)

    def outer(k, carry):
        _start(2 * k + 1, 1)
        _wait(0)
        _compute(2 * k, 0)

        @pl.when(k < UPW // 2 - 1)
        def _():
            _start(2 * k + 2, 0)

        _wait(1)
        _compute(2 * k + 1, 1)
        return carry

    lax.fori_loop(0, UPW // 2, outer, 0)

    pltpu.sync_copy(acc, out_hbm.at[wid])


def _tc_hist_body(x_ref, t_ref, o_ref):
    bi = pl.program_id(0)
    j = pl.program_id(1)

    @pl.when((bi == 0) & (j == 0))
    def _():
        o_ref[...] = jnp.zeros_like(o_ref)

    nb = DC * H // 8

    def band(hb, carry):
        xs = x_ref[0, :, pl.ds(hb * 8, 8), :]
        ts = t_ref[0, pl.ds(hb * 8, 8), :]
        best = xs[0]
        idx = jnp.zeros((8, W), jnp.int32)
        for c in range(1, C):
            v = xs[c]
            m = v > best
            best = jnp.where(m, v, best)
            idx = jnp.where(m, c, idx)
        out = []
        for c in range(1, C):
            a_tps, a_np, a_nt = carry[c - 1]
            pc = idx == c
            tc = ts == c
            a_tps = a_tps + jnp.where(pc & tc, 1.0, 0.0)
            a_np = a_np + jnp.where(pc, 1.0, 0.0)
            a_nt = a_nt + jnp.where(tc, 1.0, 0.0)
            out.append((a_tps, a_np, a_nt))
        return tuple(out)

    z = jnp.zeros((8, W), jnp.float32)
    init = tuple((z, z, z) for _ in range(1, C))
    accs = lax.fori_loop(0, nb, band, init)

    rows = lax.broadcasted_iota(jnp.int32, (16, 128), 0)
    lanes = lax.broadcasted_iota(jnp.int32, (16, 128), 1)
    acc = o_ref[...]
    for c in range(1, C):
        a_tps, a_np, a_nt = accs[c - 1]
        tps = jnp.sum(a_tps)
        npred = jnp.sum(a_np)
        nt = jnp.sum(a_nt)
        row = bi * 8 + c
        upd = jnp.where(lanes == 0, tps,
                        jnp.where(lanes == 1, npred,
                                  jnp.where(lanes == 2, nt, 0.0)))
        acc = jnp.where(rows == row, acc + upd, acc)
    o_ref[...] = acc


def _dice_body(p_ref, q_ref, o_ref):
    x = p_ref[:]
    tot = jnp.sum(x, axis=0, keepdims=True)
    l = lax.broadcasted_iota(jnp.int32, (1, ACC_LEN), 1)
    key = l // LANES
    b_l = key // (C * C)
    p_l = (key // C) % C
    t_l = key % C
    q = q_ref[:]
    eps = 1e-5
    res = []
    for b in range(B):
        mb = b_l == b
        s = 0.0
        for c in range(1, C):
            tps = (jnp.sum(jnp.where(mb & (p_l == c) & (t_l == c), tot, 0.0))
                   + q[b * 8 + c, 0])
            npred = (jnp.sum(jnp.where(mb & (p_l == c), tot, 0.0))
                     + q[b * 8 + c, 1])
            nt = (jnp.sum(jnp.where(mb & (t_l == c), tot, 0.0))
                  + q[b * 8 + c, 2])
            s = s + 2.0 * tps / (npred + nt + eps)
        res.append(s / (C - 1))
    rows = lax.broadcasted_iota(jnp.int32, (8, 128), 0)
    lanes = lax.broadcasted_iota(jnp.int32, (8, 128), 1)
    out = jnp.where((rows == 0) & (lanes == 0), res[0], 0.0)
    out = jnp.where((rows == 1) & (lanes == 0), res[1], out)
    o_ref[:] = out


def kernel(inputs, targets):
    t = targets.astype(jnp.int32)
    partial = _sc_hist(inputs, t)
    x4 = inputs.reshape(B, C, D * H, W)
    t3 = t.reshape(B, D * H, W)
    tc_counts = pl.pallas_call(
        _tc_hist_body,
        grid=(B, TC_D // DC),
        in_specs=[
            pl.BlockSpec((1, C, DC * H, W),
                         lambda b, j: (b, 0, SC_D // DC + j, 0)),
            pl.BlockSpec((1, DC * H, W),
                         lambda b, j: (b, SC_D // DC + j, 0)),
        ],
        out_specs=pl.BlockSpec((16, 128), lambda b, j: (0, 0)),
        out_shape=jax.ShapeDtypeStruct((16, 128), jnp.float32),
    )(x4, t3)
    out = pl.pallas_call(
        _dice_body,
        out_shape=jax.ShapeDtypeStruct((8, 128), jnp.float32),
    )(partial, tc_counts)
    return out[0:2, 0]
